# baseline (speedup 1.0000x reference)
import functools

import jax
import jax.numpy as jnp
from jax import lax
from jax.experimental import pallas as pl
from jax.experimental.pallas import tpu as pltpu
from jax.experimental.pallas import tpu_sc as plsc

ACTION_SIZE = 4
POINT_SIZE = 11
EMBED_DIM = 128
LANES = 16
NUM_WORKERS = 32
BATCH = 16384
ROWS_PER_W = BATCH // NUM_WORKERS
CHUNKS_PER_W = ROWS_PER_W // LANES
BLOCK_B = 2048
W_PER_BLOCK = BLOCK_B // ROWS_PER_W


def _tc_sims_body(ew_ref, av_ref, out_ref):
    ew = ew_ref[...]
    av = av_ref[0]
    num = jax.lax.dot_general(
        av, ew, (((0,), (1,)), ((), ())),
        precision=jax.lax.Precision.HIGHEST,
        preferred_element_type=jnp.float32)
    n1 = jnp.sqrt(jnp.sum(ew * ew, axis=1, keepdims=True)).T
    n2 = jnp.sqrt(jnp.sum(av * av, axis=0, keepdims=True)).T
    sims = num / jnp.maximum(n1 * n2, 1e-8)
    for j in range(W_PER_BLOCK):
        out_ref[j] = sims[:, j * ROWS_PER_W:(j + 1) * ROWS_PER_W]


def _sims_transposed(embedded_words, action_vectors):
    return pl.pallas_call(
        _tc_sims_body,
        grid=(BATCH // BLOCK_B,),
        in_specs=[
            pl.BlockSpec((BLOCK_B, EMBED_DIM), lambda i: (i, 0)),
            pl.BlockSpec((1, EMBED_DIM, POINT_SIZE), lambda i: (0, 0, 0)),
        ],
        out_specs=pl.BlockSpec((W_PER_BLOCK, POINT_SIZE, ROWS_PER_W),
                               lambda i: (i, 0, 0)),
        out_shape=jax.ShapeDtypeStruct((NUM_WORKERS, POINT_SIZE, ROWS_PER_W),
                                       jnp.float32),
    )(embedded_words, action_vectors)


@functools.partial(
    pl.kernel,
    mesh=plsc.VectorSubcoreMesh(core_axis_name="c", subcore_axis_name="s"),
    out_type=jax.ShapeDtypeStruct((BATCH, ACTION_SIZE), jnp.float32),
    scratch_types=[
        pltpu.VMEM((POINT_SIZE, ROWS_PER_W), jnp.float32),
        pltpu.VMEM((ROWS_PER_W, ACTION_SIZE), jnp.float32),
    ],
    compiler_params=pltpu.CompilerParams(needs_layout_passes=False),
)
def _sc_onehot(sims_hbm, zeros_hbm, out_hbm, sims_v, out_v):
    wid = lax.axis_index("s") * 2 + lax.axis_index("c")
    pltpu.sync_copy(zeros_hbm, out_v)
    pltpu.sync_copy(sims_hbm.at[wid], sims_v)
    ones = jnp.full((LANES,), 1.0, jnp.float32)
    i0 = jnp.zeros((LANES,), jnp.int32)
    i1 = jnp.full((LANES,), 1, jnp.int32)
    i2 = jnp.full((LANES,), 2, jnp.int32)
    i3 = jnp.full((LANES,), 3, jnp.int32)
    for i in range(CHUNKS_PER_W):
        s = [sims_v[p, pl.ds(i * LANES, LANES)] for p in range(POINT_SIZE)]
        p0 = jnp.maximum(jnp.maximum(s[0], s[1]), jnp.maximum(s[2], s[3]))
        p1 = jnp.maximum(jnp.maximum(jnp.maximum(s[4], s[5]),
                                     jnp.maximum(s[6], s[7])), s[8])
        p2 = s[9]
        p3 = s[10]
        best = jnp.maximum(jnp.maximum(p0, p1), jnp.maximum(p2, p3))
        widx = jnp.where(p0 >= best, i0,
               jnp.where(p1 >= best, i1,
               jnp.where(p2 >= best, i2, i3)))
        row_idx = lax.iota(jnp.int32, LANES) + i * LANES
        plsc.store_scatter(out_v, [row_idx, widx], ones)
    pltpu.sync_copy(out_v, out_hbm.at[pl.ds(wid * ROWS_PER_W, ROWS_PER_W)])


def kernel(embedded_words, action_vectors):
    sims = _sims_transposed(embedded_words, action_vectors)
    zeros = jnp.zeros((ROWS_PER_W, ACTION_SIZE), jnp.float32)
    return _sc_onehot(sims, zeros)


# speedup vs baseline: 2.0219x; 2.0219x over previous
import jax
import jax.numpy as jnp
from jax.experimental import pallas as pl

ACTION_SIZE = 4
POINT_SIZE = 11
EMBED_DIM = 128
P_PAD = 16

BLOCK_B = 2048


def _tc_body(ew_ref, av_ref, out_ref):
    ew = ew_ref[...]
    av = av_ref[...]
    num = jax.lax.dot_general(
        ew, av, (((1,), (0,)), ((), ())),
        precision=jax.lax.Precision.HIGHEST,
        preferred_element_type=jnp.float32)
    n1 = jnp.sqrt(jnp.sum(ew * ew, axis=1, keepdims=True))
    n2 = jnp.sqrt(jnp.sum(av * av, axis=0, keepdims=True))
    sims = num / jnp.maximum(n1 * n2, 1e-8)
    col = jax.lax.broadcasted_iota(jnp.int32, (BLOCK_B, P_PAD), 1)
    sims = jnp.where(col < POINT_SIZE, sims, -jnp.inf)
    rowmax = jnp.max(sims, axis=1, keepdims=True)
    first_p = jnp.min(jnp.where(sims == rowmax, col, P_PAD), axis=1,
                      keepdims=True)
    action = jnp.where(first_p < 4, 0,
             jnp.where(first_p < 9, 1,
             jnp.where(first_p == 9, 2, 3)))
    a4 = jax.lax.broadcasted_iota(jnp.int32, (BLOCK_B, ACTION_SIZE), 1)
    out_ref[...] = (a4 == action).astype(jnp.float32)


def kernel(embedded_words, action_vectors):
    batch = embedded_words.shape[0]
    av = jnp.pad(action_vectors[0], ((0, 0), (0, P_PAD - POINT_SIZE)))
    grid = (batch // BLOCK_B,)
    return pl.pallas_call(
        _tc_body,
        grid=grid,
        in_specs=[
            pl.BlockSpec((BLOCK_B, EMBED_DIM), lambda i: (i, 0)),
            pl.BlockSpec((EMBED_DIM, P_PAD), lambda i: (0, 0)),
        ],
        out_specs=pl.BlockSpec((BLOCK_B, ACTION_SIZE), lambda i: (i, 0)),
        out_shape=jax.ShapeDtypeStruct((batch, ACTION_SIZE), jnp.float32),
    )(embedded_words, av)
